# baseline (speedup 1.0000x reference)
import jax
import jax.numpy as jnp
from jax import lax
from jax.experimental import pallas as pl
from jax.experimental.pallas import tpu as pltpu
from jax.experimental.pallas import tpu_sc as plsc

_NW = 32
_BM = 512
_NS = 4


def _logits_body(*refs):
    x_refs = refs[:_NS]
    wt_ref, b_ref, out_ref = refs[_NS:]
    for k in range(_NS):
        out_ref[k] = lax.dot_general(
            x_refs[k][0], wt_ref[...], (((1,), (0,)), ((), ())),
            preferred_element_type=jnp.float32) + b_ref[...]


def _router_body(logits_hbm, idx_hbm, wts_hbm, lbuf, iout, wout):
    tokens = logits_hbm.shape[0]
    experts = logits_hbm.shape[1]
    chunk = tokens // _NW
    seq = idx_hbm.shape[1]
    wid = lax.axis_index("s") * 2 + lax.axis_index("c")
    base = wid * chunk
    pltpu.sync_copy(logits_hbm.at[pl.ds(base, chunk)], lbuf)
    lane = lax.iota(jnp.int32, 16)

    def group(g, carry):
        rows = lane + g * 16
        zero = jnp.zeros((16,), jnp.int32)
        one = jnp.ones((16,), jnp.int32)
        v0 = plsc.load_gather(lbuf, [rows, zero])
        v1 = plsc.load_gather(lbuf, [rows, one])
        sw = v1 > v0
        m1 = jnp.where(sw, v1, v0)
        i1 = jnp.where(sw, one, zero)
        m2 = jnp.where(sw, v0, v1)
        i2 = jnp.where(sw, zero, one)
        for e in range(2, experts):
            ev = jnp.full((16,), e, jnp.int32)
            v = plsc.load_gather(lbuf, [rows, ev])
            gt1 = v > m1
            gt2 = v > m2
            m2 = jnp.where(gt1, m1, jnp.where(gt2, v, m2))
            i2 = jnp.where(gt1, i1, jnp.where(gt2, ev, i2))
            m1 = jnp.where(gt1, v, m1)
            i1 = jnp.where(gt1, ev, i1)
        ex = jnp.exp(m2 - m1)
        w1 = 1.0 / (1.0 + ex)
        plsc.store_scatter(iout, [rows, zero], i1)
        plsc.store_scatter(iout, [rows, one], i2)
        plsc.store_scatter(wout, [rows, zero], w1)
        plsc.store_scatter(wout, [rows, one], 1.0 - w1)
        return carry

    lax.fori_loop(0, chunk // 16, group, 0)
    b0 = base // seq
    r0 = base % seq
    pltpu.sync_copy(iout, idx_hbm.at[b0, pl.ds(r0, chunk)])
    pltpu.sync_copy(wout, wts_hbm.at[b0, pl.ds(r0, chunk)])


def kernel(x, W, b):
    batch, seq, feat = x.shape
    tokens = batch * seq
    experts = W.shape[0]
    rows = tokens // _NS
    xs = x.reshape(_NS, rows, feat)

    def x_spec(k):
        return pl.BlockSpec((1, _BM, feat), lambda i, k=k: (k, i, 0))

    logits = pl.pallas_call(
        _logits_body,
        grid=(rows // _BM,),
        in_specs=(
            [x_spec(k) for k in range(_NS)]
            + [pl.BlockSpec((feat, experts), lambda i: (0, 0)),
               pl.BlockSpec((1, experts), lambda i: (0, 0))]
        ),
        out_specs=pl.BlockSpec((_NS, _BM, experts), lambda i: (0, i, 0)),
        out_shape=jax.ShapeDtypeStruct((_NS, rows, experts), jnp.float32),
    )(*([xs] * _NS), W.T, b.reshape(1, experts))

    chunk = tokens // _NW
    router = pl.kernel(
        _router_body,
        out_type=(jax.ShapeDtypeStruct((batch, seq, 2), jnp.int32),
                  jax.ShapeDtypeStruct((batch, seq, 2), jnp.float32)),
        mesh=plsc.VectorSubcoreMesh(core_axis_name="c", subcore_axis_name="s"),
        compiler_params=pltpu.CompilerParams(
            needs_layout_passes=False, use_tc_tiling_on_sc=False),
        scratch_types=[
            pltpu.VMEM((chunk, experts), jnp.float32),
            pltpu.VMEM((chunk, 2), jnp.int32),
            pltpu.VMEM((chunk, 2), jnp.float32),
        ],
    )
    idx, wts = router(logits.reshape(tokens, experts))
    return idx, wts


# speedup vs baseline: 1.0148x; 1.0148x over previous
import jax
import jax.numpy as jnp
from jax import lax
from jax.experimental import pallas as pl
from jax.experimental.pallas import tpu as pltpu
from jax.experimental.pallas import tpu_sc as plsc

_NW = 32
_BM = 1024


def _logits_body(x_ref, wt_ref, b_ref, out_ref):
    out_ref[...] = lax.dot_general(
        x_ref[...], wt_ref[...], (((1,), (0,)), ((), ())),
        preferred_element_type=jnp.float32) + b_ref[...]


def _router_body(logits_hbm, idx_hbm, wts_hbm, lbuf, iout, wout):
    tokens = logits_hbm.shape[0]
    experts = logits_hbm.shape[1]
    chunk = tokens // _NW
    wid = lax.axis_index("s") * 2 + lax.axis_index("c")
    base = wid * chunk
    pltpu.sync_copy(logits_hbm.at[pl.ds(base, chunk)], lbuf)
    lane = lax.iota(jnp.int32, 16)

    def group(g, carry):
        rows = lane + g * 16
        zero = jnp.zeros((16,), jnp.int32)
        one = jnp.ones((16,), jnp.int32)
        v0 = plsc.load_gather(lbuf, [rows, zero])
        v1 = plsc.load_gather(lbuf, [rows, one])
        sw = v1 > v0
        m1 = jnp.where(sw, v1, v0)
        i1 = jnp.where(sw, one, zero)
        m2 = jnp.where(sw, v0, v1)
        i2 = jnp.where(sw, zero, one)
        for e in range(2, experts):
            ev = jnp.full((16,), e, jnp.int32)
            v = plsc.load_gather(lbuf, [rows, ev])
            gt1 = v > m1
            gt2 = v > m2
            m2 = jnp.where(gt1, m1, jnp.where(gt2, v, m2))
            i2 = jnp.where(gt1, i1, jnp.where(gt2, ev, i2))
            m1 = jnp.where(gt1, v, m1)
            i1 = jnp.where(gt1, ev, i1)
        ex = jnp.exp(m2 - m1)
        w1 = 1.0 / (1.0 + ex)
        plsc.store_scatter(iout, [rows, zero], i1)
        plsc.store_scatter(iout, [rows, one], i2)
        plsc.store_scatter(wout, [rows, zero], w1)
        plsc.store_scatter(wout, [rows, one], 1.0 - w1)
        return carry

    lax.fori_loop(0, chunk // 16, group, 0)
    pltpu.sync_copy(iout, idx_hbm.at[pl.ds(base, chunk)])
    pltpu.sync_copy(wout, wts_hbm.at[pl.ds(base, chunk)])


def kernel(x, W, b):
    batch, seq, feat = x.shape
    tokens = batch * seq
    experts = W.shape[0]
    xf = x.reshape(tokens, feat)

    logits = pl.pallas_call(
        _logits_body,
        grid=(tokens // _BM,),
        in_specs=[
            pl.BlockSpec((_BM, feat), lambda i: (i, 0)),
            pl.BlockSpec((feat, experts), lambda i: (0, 0)),
            pl.BlockSpec((1, experts), lambda i: (0, 0)),
        ],
        out_specs=pl.BlockSpec((_BM, experts), lambda i: (i, 0)),
        out_shape=jax.ShapeDtypeStruct((tokens, experts), jnp.float32),
    )(xf, W.T, b.reshape(1, experts))

    chunk = tokens // _NW
    router = pl.kernel(
        _router_body,
        out_type=(jax.ShapeDtypeStruct((tokens, 2), jnp.int32),
                  jax.ShapeDtypeStruct((tokens, 2), jnp.float32)),
        mesh=plsc.VectorSubcoreMesh(core_axis_name="c", subcore_axis_name="s"),
        compiler_params=pltpu.CompilerParams(
            needs_layout_passes=False, use_tc_tiling_on_sc=False),
        scratch_types=[
            pltpu.VMEM((chunk, experts), jnp.float32),
            pltpu.VMEM((chunk, 2), jnp.int32),
            pltpu.VMEM((chunk, 2), jnp.float32),
        ],
    )
    idx, wts = router(logits)
    return idx.reshape(batch, seq, 2), wts.reshape(batch, seq, 2)
